# baseline (speedup 1.0000x reference)
import functools

import jax
import jax.numpy as jnp
from jax import lax
from jax.experimental import pallas as pl
from jax.experimental.pallas import tpu as pltpu
from jax.experimental.pallas import tpu_sc as plsc

VOCAB = 100000
EMBED_DIM = 1024
MAX_POS = 8192
BATCH = 4
SEQLEN = 8192

NC = 2
NS = 16
NW = NC * NS

N_ROWS = BATCH * SEQLEN
ROWS_PER_W = N_ROWS // NW
CHUNK = 32
N_CHUNKS = ROWS_PER_W // CHUNK


UNROLL = 8
VECS_PER_ROW = EMBED_DIM // 16


def _sc_body(ids_hbm, wt_hbm, pt_hbm, out_hbm, idx_v, buf_w, buf_p, sem):
    wid = lax.axis_index("s") * NC + lax.axis_index("c")
    base = wid * ROWS_PER_W
    pos_base = lax.rem(base, SEQLEN)

    pltpu.sync_copy(ids_hbm.at[wid], idx_v)

    def chunk_body(i, carry):
        r0 = i * CHUNK
        gather = pltpu.async_copy(wt_hbm.at[idx_v.at[i]], buf_w, sem)
        pltpu.sync_copy(pt_hbm.at[pl.ds(pos_base + r0, CHUNK)], buf_p)
        gather.wait()

        def add_body(r, c2):
            def col_body(j, c3):
                for u in range(UNROLL):
                    col = (j * UNROLL + u) * 16
                    buf_w[r, pl.ds(col, 16)] = (
                        buf_w[r, pl.ds(col, 16)] + buf_p[r, pl.ds(col, 16)]
                    )
                return c3

            return lax.fori_loop(0, VECS_PER_ROW // UNROLL, col_body, c2)

        lax.fori_loop(0, CHUNK, add_body, 0)

        pltpu.sync_copy(buf_w, out_hbm.at[pl.ds(base + r0, CHUNK)])
        return carry

    lax.fori_loop(0, N_CHUNKS, chunk_body, 0)


@jax.jit
def _embed(ids_flat, word_table, pos_table):
    mesh = plsc.VectorSubcoreMesh(
        core_axis_name="c", subcore_axis_name="s", num_cores=NC, num_subcores=NS
    )
    kfn = pl.kernel(
        _sc_body,
        out_type=jax.ShapeDtypeStruct((N_ROWS, EMBED_DIM), jnp.float32),
        mesh=mesh,
        scratch_types=[
            pltpu.VMEM((N_CHUNKS, CHUNK), jnp.int32),
            pltpu.VMEM((CHUNK, EMBED_DIM), jnp.float32),
            pltpu.VMEM((CHUNK, EMBED_DIM), jnp.float32),
            pltpu.SemaphoreType.DMA,
        ],
    )
    return kfn(ids_flat, word_table, pos_table)


def kernel(input_ids, word_table, pos_table):
    ids = input_ids.astype(jnp.int32).reshape(NW, N_CHUNKS, CHUNK)
    out = _embed(ids, word_table, pos_table)
    return out.reshape(BATCH, SEQLEN, EMBED_DIM)


# speedup vs baseline: 2.1804x; 2.1804x over previous
import jax
import jax.numpy as jnp
from jax import lax
from jax.experimental import pallas as pl
from jax.experimental.pallas import tpu as pltpu
from jax.experimental.pallas import tpu_sc as plsc

VOCAB = 100000
EMBED_DIM = 1024
MAX_POS = 8192
BATCH = 4
SEQLEN = 8192

NC = 2
NS = 16
NW = NC * NS

N_ROWS = BATCH * SEQLEN
ROWS_PER_W = N_ROWS // NW
CHUNK = 16
N_CHUNKS = ROWS_PER_W // CHUNK
NSW = 4
NSP = 2

UNROLL = 8
VECS_PER_ROW = EMBED_DIM // 16


def _sc_body(ids_hbm, wt_hbm, pt_hbm, out_hbm, idx_v, buf_w, buf_p, sem_g, sem_p, sem_s):
    wid = lax.axis_index("s") * NC + lax.axis_index("c")
    base = wid * ROWS_PER_W
    pos_base = lax.rem(base, SEQLEN)

    pltpu.sync_copy(ids_hbm.at[wid], idx_v)

    def gather_copy(i):
        return pltpu.make_async_copy(
            wt_hbm.at[idx_v.at[i]], buf_w.at[lax.rem(i, NSW)], sem_g.at[lax.rem(i, NSW)]
        )

    def pos_copy(i):
        return pltpu.make_async_copy(
            pt_hbm.at[pl.ds(pos_base + i * CHUNK, CHUNK)],
            buf_p.at[lax.rem(i, NSP)],
            sem_p.at[lax.rem(i, NSP)],
        )

    def store_copy(i):
        return pltpu.make_async_copy(
            buf_w.at[lax.rem(i, NSW)],
            out_hbm.at[pl.ds(base + i * CHUNK, CHUNK)],
            sem_s.at[lax.rem(i, NSW)],
        )

    gather_copy(0).start()
    gather_copy(1).start()
    pos_copy(0).start()

    def chunk_body(i, carry):
        sw = lax.rem(i, NSW)
        sp = lax.rem(i, NSP)
        gather_copy(i).wait()
        pos_copy(i).wait()

        @pl.when(i + 1 < N_CHUNKS)
        def _():
            pos_copy(i + 1).start()

        def add_row(r, c2):
            def add_cols(j, c3):
                for u in range(UNROLL):
                    col = (j * UNROLL + u) * 16
                    plsc.addupdate(
                        buf_w.at[sw, r, pl.ds(col, 16)],
                        buf_p[sp, r, pl.ds(col, 16)],
                    )
                return c3

            return lax.fori_loop(0, VECS_PER_ROW // UNROLL, add_cols, c2)

        lax.fori_loop(0, CHUNK, add_row, 0)

        store_copy(i).start()

        @pl.when(i + 2 < N_CHUNKS)
        def _():
            @pl.when(i >= 2)
            def _():
                store_copy(i - 2).wait()

            gather_copy(i + 2).start()

        return carry

    lax.fori_loop(0, N_CHUNKS, chunk_body, 0)

    for j in range(max(0, N_CHUNKS - 4), N_CHUNKS):
        store_copy(j).wait()


@jax.jit
def _embed(ids_flat, word_table, pos_table):
    mesh = plsc.VectorSubcoreMesh(
        core_axis_name="c", subcore_axis_name="s", num_cores=NC, num_subcores=NS
    )
    kfn = pl.kernel(
        _sc_body,
        out_type=jax.ShapeDtypeStruct((N_ROWS, EMBED_DIM), jnp.float32),
        mesh=mesh,
        scratch_types=[
            pltpu.VMEM((N_CHUNKS, CHUNK), jnp.int32),
            pltpu.VMEM((NSW, CHUNK, EMBED_DIM), jnp.float32),
            pltpu.VMEM((NSP, CHUNK, EMBED_DIM), jnp.float32),
            pltpu.SemaphoreType.DMA((NSW,)),
            pltpu.SemaphoreType.DMA((NSP,)),
            pltpu.SemaphoreType.DMA((NSW,)),
        ],
    )
    return kfn(ids_flat, word_table, pos_table)


def kernel(input_ids, word_table, pos_table):
    ids = input_ids.astype(jnp.int32).reshape(NW, N_CHUNKS, CHUNK)
    out = _embed(ids, word_table, pos_table)
    return out.reshape(BATCH, SEQLEN, EMBED_DIM)


# speedup vs baseline: 3.3349x; 1.5295x over previous
import jax
import jax.numpy as jnp
from jax import lax
from jax.experimental import pallas as pl
from jax.experimental.pallas import tpu as pltpu
from jax.experimental.pallas import tpu_sc as plsc

VOCAB = 100000
EMBED_DIM = 1024
MAX_POS = 8192
BATCH = 4
SEQLEN = 8192

NC = 2
NS = 16
NW = NC * NS

N_ROWS = BATCH * SEQLEN
ROWS_PER_W = N_ROWS // NW
CHUNK = 16
N_CHUNKS = ROWS_PER_W // CHUNK
NSW = 4
NSP = 2

UNROLL = 8
VECS_PER_ROW = EMBED_DIM // 16


def _sc_body(ids_hbm, wt_hbm, pt_hbm, out_hbm, idx_v, buf_w, buf_p, sem_g, sem_p, sem_s):
    wid = lax.axis_index("s") * NC + lax.axis_index("c")
    base = wid * ROWS_PER_W
    pos_base = lax.rem(base, SEQLEN)

    pltpu.sync_copy(ids_hbm.at[wid], idx_v)

    def gather_copy(i):
        return pltpu.make_async_copy(
            wt_hbm.at[idx_v.at[i]], buf_w.at[lax.rem(i, NSW)], sem_g.at[lax.rem(i, NSW)]
        )

    def pos_copy(i):
        return pltpu.make_async_copy(
            pt_hbm.at[pl.ds(pos_base + i * CHUNK, CHUNK)],
            buf_p.at[lax.rem(i, NSP)],
            sem_p.at[lax.rem(i, NSP)],
        )

    def store_copy(i):
        return pltpu.make_async_copy(
            buf_w.at[lax.rem(i, NSW)],
            out_hbm.at[pl.ds(base + i * CHUNK, CHUNK)],
            sem_s.at[lax.rem(i, NSW)],
        )

    gather_copy(0).start()
    gather_copy(1).start()
    pos_copy(0).start()

    def chunk_body(i, carry):
        sw = lax.rem(i, NSW)
        sp = lax.rem(i, NSP)
        gather_copy(i).wait()
        pos_copy(i).wait()

        @pl.when(i + 1 < N_CHUNKS)
        def _():
            pos_copy(i + 1).start()

        def add_row(r, c2):
            @plsc.parallel_loop(0, EMBED_DIM, step=16, unroll=UNROLL)
            def _(c):
                col = pl.multiple_of(c, 16)
                plsc.addupdate(
                    buf_w.at[sw, r, pl.ds(col, 16)],
                    buf_p[sp, r, pl.ds(col, 16)],
                )

            return c2

        lax.fori_loop(0, CHUNK, add_row, 0)

        store_copy(i).start()

        @pl.when(i + 2 < N_CHUNKS)
        def _():
            @pl.when(i >= 2)
            def _():
                store_copy(i - 2).wait()

            gather_copy(i + 2).start()

        return carry

    lax.fori_loop(0, N_CHUNKS, chunk_body, 0)

    for j in range(max(0, N_CHUNKS - 4), N_CHUNKS):
        store_copy(j).wait()


@jax.jit
def _embed(ids_flat, word_table, pos_table):
    mesh = plsc.VectorSubcoreMesh(
        core_axis_name="c", subcore_axis_name="s", num_cores=NC, num_subcores=NS
    )
    kfn = pl.kernel(
        _sc_body,
        out_type=jax.ShapeDtypeStruct((N_ROWS, EMBED_DIM), jnp.float32),
        mesh=mesh,
        scratch_types=[
            pltpu.VMEM((N_CHUNKS, CHUNK), jnp.int32),
            pltpu.VMEM((NSW, CHUNK, EMBED_DIM), jnp.float32),
            pltpu.VMEM((NSP, CHUNK, EMBED_DIM), jnp.float32),
            pltpu.SemaphoreType.DMA((NSW,)),
            pltpu.SemaphoreType.DMA((NSP,)),
            pltpu.SemaphoreType.DMA((NSW,)),
        ],
    )
    return kfn(ids_flat, word_table, pos_table)


def kernel(input_ids, word_table, pos_table):
    ids = input_ids.astype(jnp.int32).reshape(NW, N_CHUNKS, CHUNK)
    out = _embed(ids, word_table, pos_table)
    return out.reshape(BATCH, SEQLEN, EMBED_DIM)


# speedup vs baseline: 4.0827x; 1.2242x over previous
import jax
import jax.numpy as jnp
from jax import lax
from jax.experimental import pallas as pl
from jax.experimental.pallas import tpu as pltpu
from jax.experimental.pallas import tpu_sc as plsc

VOCAB = 100000
EMBED_DIM = 1024
MAX_POS = 8192
BATCH = 4
SEQLEN = 8192

NC = 2
NS = 16
NW = NC * NS

N_ROWS = BATCH * SEQLEN
ROWS_PER_W = N_ROWS // NW
CHUNK = 16
N_CHUNKS = ROWS_PER_W // CHUNK
SEQ_PER_W = SEQLEN // NW
POS_CHUNKS = SEQ_PER_W // CHUNK
NSW = 4
NSP = 2

UNROLL = 8
VECS_PER_ROW = EMBED_DIM // 16


def _sc_body(ids_hbm, wt_hbm, pt_hbm, out_hbm, idx_v, buf_w, buf_p, sem_g, sem_p, sem_s):
    wid = lax.axis_index("s") * NC + lax.axis_index("c")
    seq0 = wid * SEQ_PER_W

    pltpu.sync_copy(ids_hbm.at[wid], idx_v)

    def gather_copy(i):
        return pltpu.make_async_copy(
            wt_hbm.at[idx_v.at[i]], buf_w.at[lax.rem(i, NSW)], sem_g.at[lax.rem(i, NSW)]
        )

    def pos_copy(c):
        return pltpu.make_async_copy(
            pt_hbm.at[pl.ds(seq0 + c * CHUNK, CHUNK)],
            buf_p.at[lax.rem(c, NSP)],
            sem_p.at[lax.rem(c, NSP)],
        )

    def store_copy(i):
        c = lax.div(i, BATCH)
        b = lax.rem(i, BATCH)
        return pltpu.make_async_copy(
            buf_w.at[lax.rem(i, NSW)],
            out_hbm.at[pl.ds(b * SEQLEN + seq0 + c * CHUNK, CHUNK)],
            sem_s.at[lax.rem(i, NSW)],
        )

    gather_copy(0).start()
    gather_copy(1).start()
    pos_copy(0).start()

    def chunk_body(i, carry):
        sw = lax.rem(i, NSW)
        c = lax.div(i, BATCH)
        b = lax.rem(i, BATCH)
        sp = lax.rem(c, NSP)
        gather_copy(i).wait()

        @pl.when(b == 0)
        def _():
            pos_copy(c).wait()

        @pl.when((b == 2) & (c + 1 < POS_CHUNKS))
        def _():
            pos_copy(c + 1).start()

        def add_row(r, c2):
            @plsc.parallel_loop(0, EMBED_DIM, step=16, unroll=UNROLL)
            def _(col0):
                col = pl.multiple_of(col0, 16)
                plsc.addupdate(
                    buf_w.at[sw, r, pl.ds(col, 16)],
                    buf_p[sp, r, pl.ds(col, 16)],
                )

            return c2

        lax.fori_loop(0, CHUNK, add_row, 0)

        store_copy(i).start()

        @pl.when(i + 2 < N_CHUNKS)
        def _():
            @pl.when(i >= 2)
            def _():
                store_copy(i - 2).wait()

            gather_copy(i + 2).start()

        return carry

    lax.fori_loop(0, N_CHUNKS, chunk_body, 0)

    for j in range(max(0, N_CHUNKS - 4), N_CHUNKS):
        store_copy(j).wait()


@jax.jit
def _embed(ids_flat, word_table, pos_table):
    mesh = plsc.VectorSubcoreMesh(
        core_axis_name="c", subcore_axis_name="s", num_cores=NC, num_subcores=NS
    )
    kfn = pl.kernel(
        _sc_body,
        out_type=jax.ShapeDtypeStruct((N_ROWS, EMBED_DIM), jnp.float32),
        mesh=mesh,
        scratch_types=[
            pltpu.VMEM((N_CHUNKS, CHUNK), jnp.int32),
            pltpu.VMEM((NSW, CHUNK, EMBED_DIM), jnp.float32),
            pltpu.VMEM((NSP, CHUNK, EMBED_DIM), jnp.float32),
            pltpu.SemaphoreType.DMA((NSW,)),
            pltpu.SemaphoreType.DMA((NSP,)),
            pltpu.SemaphoreType.DMA((NSW,)),
        ],
    )
    return kfn(ids_flat, word_table, pos_table)


def kernel(input_ids, word_table, pos_table):
    ids = (
        input_ids.astype(jnp.int32)
        .reshape(BATCH, NW, POS_CHUNKS, CHUNK)
        .transpose(1, 2, 0, 3)
        .reshape(NW, N_CHUNKS, CHUNK)
    )
    out = _embed(ids, word_table, pos_table)
    return out.reshape(BATCH, SEQLEN, EMBED_DIM)


# speedup vs baseline: 4.3051x; 1.0545x over previous
import jax
import jax.numpy as jnp
from jax import lax
from jax.experimental import pallas as pl
from jax.experimental.pallas import tpu as pltpu
from jax.experimental.pallas import tpu_sc as plsc

VOCAB = 100000
EMBED_DIM = 1024
MAX_POS = 8192
BATCH = 4
SEQLEN = 8192

NC = 2
NS = 16
NW = NC * NS

N_ROWS = BATCH * SEQLEN
ROWS_PER_W = N_ROWS // NW
CHUNK = 16
N_CHUNKS = ROWS_PER_W // CHUNK
SEQ_PER_W = SEQLEN // NW
POS_CHUNKS = SEQ_PER_W // CHUNK
NSW = 5
NSP = 2
AHEAD = NSW - 2

UNROLL = 8
VECS_PER_ROW = EMBED_DIM // 16


def _sc_body(ids_hbm, wt_hbm, pt_hbm, out_hbm, idx_v, buf_w, buf_p, sem_g, sem_p, sem_s):
    wid = lax.axis_index("s") * NC + lax.axis_index("c")
    seq0 = wid * SEQ_PER_W

    pltpu.sync_copy(ids_hbm.at[wid], idx_v)

    def gather_copy(i):
        return pltpu.make_async_copy(
            wt_hbm.at[idx_v.at[i]], buf_w.at[lax.rem(i, NSW)], sem_g.at[lax.rem(i, NSW)]
        )

    def pos_copy(c):
        return pltpu.make_async_copy(
            pt_hbm.at[pl.ds(seq0 + c * CHUNK, CHUNK)],
            buf_p.at[lax.rem(c, NSP)],
            sem_p.at[lax.rem(c, NSP)],
        )

    def store_copy(i):
        c = lax.div(i, BATCH)
        b = lax.rem(i, BATCH)
        return pltpu.make_async_copy(
            buf_w.at[lax.rem(i, NSW)],
            out_hbm.at[pl.ds(b * SEQLEN + seq0 + c * CHUNK, CHUNK)],
            sem_s.at[lax.rem(i, NSW)],
        )

    for j in range(AHEAD):
        gather_copy(j).start()
    pos_copy(0).start()

    def chunk_body(i, carry):
        sw = lax.rem(i, NSW)
        c = lax.div(i, BATCH)
        b = lax.rem(i, BATCH)
        sp = lax.rem(c, NSP)
        gather_copy(i).wait()

        @pl.when(b == 0)
        def _():
            pos_copy(c).wait()

        @pl.when((b == 2) & (c + 1 < POS_CHUNKS))
        def _():
            pos_copy(c + 1).start()

        def add_row(r, c2):
            @plsc.parallel_loop(0, EMBED_DIM, step=16, unroll=UNROLL)
            def _(col0):
                col = pl.multiple_of(col0, 16)
                plsc.addupdate(
                    buf_w.at[sw, r, pl.ds(col, 16)],
                    buf_p[sp, r, pl.ds(col, 16)],
                )

            return c2

        lax.fori_loop(0, CHUNK, add_row, 0)

        store_copy(i).start()

        @pl.when(i + AHEAD < N_CHUNKS)
        def _():
            @pl.when(i + AHEAD >= NSW)
            def _():
                store_copy(i + AHEAD - NSW).wait()

            gather_copy(i + AHEAD).start()

        return carry

    lax.fori_loop(0, N_CHUNKS, chunk_body, 0)

    for j in range(max(0, N_CHUNKS - NSW), N_CHUNKS):
        store_copy(j).wait()


@jax.jit
def _embed(ids_flat, word_table, pos_table):
    mesh = plsc.VectorSubcoreMesh(
        core_axis_name="c", subcore_axis_name="s", num_cores=NC, num_subcores=NS
    )
    kfn = pl.kernel(
        _sc_body,
        out_type=jax.ShapeDtypeStruct((N_ROWS, EMBED_DIM), jnp.float32),
        mesh=mesh,
        scratch_types=[
            pltpu.VMEM((N_CHUNKS, CHUNK), jnp.int32),
            pltpu.VMEM((NSW, CHUNK, EMBED_DIM), jnp.float32),
            pltpu.VMEM((NSP, CHUNK, EMBED_DIM), jnp.float32),
            pltpu.SemaphoreType.DMA((NSW,)),
            pltpu.SemaphoreType.DMA((NSP,)),
            pltpu.SemaphoreType.DMA((NSW,)),
        ],
    )
    return kfn(ids_flat, word_table, pos_table)


def kernel(input_ids, word_table, pos_table):
    ids = (
        input_ids.astype(jnp.int32)
        .reshape(BATCH, NW, POS_CHUNKS, CHUNK)
        .transpose(1, 2, 0, 3)
        .reshape(NW, N_CHUNKS, CHUNK)
    )
    out = _embed(ids, word_table, pos_table)
    return out.reshape(BATCH, SEQLEN, EMBED_DIM)


# speedup vs baseline: 4.3459x; 1.0095x over previous
import jax
import jax.numpy as jnp
from jax import lax
from jax.experimental import pallas as pl
from jax.experimental.pallas import tpu as pltpu
from jax.experimental.pallas import tpu_sc as plsc

VOCAB = 100000
EMBED_DIM = 1024
MAX_POS = 8192
BATCH = 4
SEQLEN = 8192

NC = 2
NS = 16
NW = NC * NS

N_ROWS = BATCH * SEQLEN
ROWS_PER_W = N_ROWS // NW
CHUNK = 16
N_CHUNKS = ROWS_PER_W // CHUNK
SEQ_PER_W = SEQLEN // NW
POS_CHUNKS = SEQ_PER_W // CHUNK
NSW = 5
NSP = 2
AHEAD = NSW - 2

UNROLL = 8
VECS_PER_ROW = EMBED_DIM // 16


def _sc_body(ids_hbm, wt_hbm, pt_hbm, out_hbm, idx_v, buf_w, buf_p, sem_g, sem_p, sem_s):
    wid = lax.axis_index("s") * NC + lax.axis_index("c")
    seq0 = wid * SEQ_PER_W

    for b in range(BATCH):
        pltpu.make_async_copy(
            ids_hbm.at[b, pl.ds(seq0, SEQ_PER_W)], idx_v.at[b], sem_g.at[b % NSW]
        ).start()
    for b in range(BATCH):
        pltpu.make_async_copy(
            ids_hbm.at[b, pl.ds(seq0, SEQ_PER_W)], idx_v.at[b], sem_g.at[b % NSW]
        ).wait()

    def gather_copy(i):
        c = lax.div(i, BATCH)
        b = lax.rem(i, BATCH)
        return pltpu.make_async_copy(
            wt_hbm.at[idx_v.at[b, pl.ds(c * CHUNK, CHUNK)]],
            buf_w.at[lax.rem(i, NSW)],
            sem_g.at[lax.rem(i, NSW)],
        )

    def pos_copy(c):
        return pltpu.make_async_copy(
            pt_hbm.at[pl.ds(seq0 + c * CHUNK, CHUNK)],
            buf_p.at[lax.rem(c, NSP)],
            sem_p.at[lax.rem(c, NSP)],
        )

    def store_copy(i):
        c = lax.div(i, BATCH)
        b = lax.rem(i, BATCH)
        return pltpu.make_async_copy(
            buf_w.at[lax.rem(i, NSW)],
            out_hbm.at[pl.ds(b * SEQLEN + seq0 + c * CHUNK, CHUNK)],
            sem_s.at[lax.rem(i, NSW)],
        )

    for j in range(AHEAD):
        gather_copy(j).start()
    pos_copy(0).start()

    def chunk_body(i, carry):
        sw = lax.rem(i, NSW)
        c = lax.div(i, BATCH)
        b = lax.rem(i, BATCH)
        sp = lax.rem(c, NSP)
        gather_copy(i).wait()

        @pl.when(b == 0)
        def _():
            pos_copy(c).wait()

        @pl.when((b == 2) & (c + 1 < POS_CHUNKS))
        def _():
            pos_copy(c + 1).start()

        def add_row(r, c2):
            @plsc.parallel_loop(0, EMBED_DIM, step=16, unroll=UNROLL)
            def _(col0):
                col = pl.multiple_of(col0, 16)
                plsc.addupdate(
                    buf_w.at[sw, r, pl.ds(col, 16)],
                    buf_p[sp, r, pl.ds(col, 16)],
                )

            return c2

        lax.fori_loop(0, CHUNK, add_row, 0)

        store_copy(i).start()

        @pl.when(i + AHEAD < N_CHUNKS)
        def _():
            @pl.when(i + AHEAD >= NSW)
            def _():
                store_copy(i + AHEAD - NSW).wait()

            gather_copy(i + AHEAD).start()

        return carry

    lax.fori_loop(0, N_CHUNKS, chunk_body, 0)

    for j in range(max(0, N_CHUNKS - NSW), N_CHUNKS):
        store_copy(j).wait()


@jax.jit
def _embed(ids_flat, word_table, pos_table):
    mesh = plsc.VectorSubcoreMesh(
        core_axis_name="c", subcore_axis_name="s", num_cores=NC, num_subcores=NS
    )
    kfn = pl.kernel(
        _sc_body,
        out_type=jax.ShapeDtypeStruct((N_ROWS, EMBED_DIM), jnp.float32),
        mesh=mesh,
        scratch_types=[
            pltpu.VMEM((BATCH, SEQ_PER_W), jnp.int32),
            pltpu.VMEM((NSW, CHUNK, EMBED_DIM), jnp.float32),
            pltpu.VMEM((NSP, CHUNK, EMBED_DIM), jnp.float32),
            pltpu.SemaphoreType.DMA((NSW,)),
            pltpu.SemaphoreType.DMA((NSP,)),
            pltpu.SemaphoreType.DMA((NSW,)),
        ],
    )
    return kfn(ids_flat, word_table, pos_table)


def kernel(input_ids, word_table, pos_table):
    out = _embed(input_ids.astype(jnp.int32), word_table, pos_table)
    return out.reshape(BATCH, SEQLEN, EMBED_DIM)
